# baseline (device time: 231981 ns/iter reference)
import jax
import jax.numpy as jnp
from jax import lax
from jax.experimental import pallas as pl
from jax.experimental.pallas import tpu as pltpu

M = 8192
D = 2048
QROWS = M // 4
N_CH = 4
CH = QROWS // N_CH
N_YDIR = 2
CX_CHUNK = 2
CZ_CHUNK = 3


def _fused(mysend_bf, slab, gamma):

    def body(mysend_ref, slab_ref, gamma_ref, out_ref, other_vm,
             y_s, y_r, a_s, a_r, b_s, b_r, cx_s, cx_r, cz_s, cz_r,
             slab_v, out_v, dsem_s, dsem_w):
        x = lax.axis_index("x")
        yy = lax.axis_index("y")
        z = lax.axis_index("z")
        dev_y = (x, 1 - yy, z)
        dev_x = (1 - x, yy, z)
        dev_z = (x, yy, 1 - z)
        qm = 2 * x + z
        qx = 2 * (1 - x) + z
        qz = 2 * x + (1 - z)
        qd = 2 * (1 - x) + (1 - z)

        barrier = pltpu.get_barrier_semaphore()
        for d in (dev_y, dev_x, dev_z):
            pl.semaphore_signal(
                barrier, inc=1, device_id=d,
                device_id_type=pl.DeviceIdType.MESH,
            )
        pl.semaphore_wait(barrier, 3)

        def rows(q, c):
            return pl.ds(q * QROWS + c * CH, CH)

        def copy(src, dst, ssem, rsem, dev):
            return pltpu.make_async_remote_copy(
                src_ref=src, dst_ref=dst, send_sem=ssem, recv_sem=rsem,
                device_id=dev, device_id_type=pl.DeviceIdType.MESH,
            )

        state = {"pending": None, "slot": 0}
        out_pending = [None, None]

        def _stage(off):
            s = state["slot"]
            state["slot"] = 1 - s
            cp = pltpu.make_async_copy(slab_ref.at[pl.ds(off, CH)],
                                       slab_v.at[s], dsem_s.at[s])
            cp.start()
            return (off, cp, s)

        def _flush(p):
            off, cp, s = p
            cp.wait()
            if out_pending[s] is not None:
                out_pending[s].wait()
            yv = slab_v[s] + other_vm[pl.ds(off, CH)].astype(jnp.float32)
            ms = (yv * yv).sum(axis=1, keepdims=True) * (1.0 / D) + 1e-6
            out_v[s] = yv * lax.rsqrt(ms) * gamma_ref[...]
            cpo = pltpu.make_async_copy(out_v.at[s],
                                        out_ref.at[pl.ds(off, CH)],
                                        dsem_w.at[s])
            cpo.start()
            out_pending[s] = cpo

        def compute_block(off):
            p = _stage(off)
            if state["pending"] is not None:
                _flush(state["pending"])
            state["pending"] = p

        sends = []

        y_rdmas = []
        for c in range(N_CH):
            r = copy(mysend_ref.at[pl.ds(c * CH, CH)],
                     other_vm.at[rows(qm, c)],
                     y_s.at[c], y_r.at[c], dev_y)
            r.start()
            y_rdmas.append(r)
            sends.append(r)
        for c in range(N_YDIR):
            r = copy(mysend_ref.at[pl.ds((N_CH + c) * CH, CH)],
                     other_vm.at[rows(qd, c)],
                     y_s.at[N_CH + c], y_r.at[N_CH + c], dev_y)
            r.start()
            y_rdmas.append(r)
            sends.append(r)

        for c in range(N_CH):
            y_rdmas[c].wait_recv()
            for dev, ss, rr in ((dev_x, a_s, a_r), (dev_z, b_s, b_r)):
                r = copy(other_vm.at[rows(qm, c)], other_vm.at[rows(qm, c)],
                         ss.at[c], rr.at[c], dev)
                r.start()
                sends.append(r)
            compute_block(qm * QROWS + c * CH)

        for c in range(N_CH):
            copy(other_vm.at[rows(qx, c)], other_vm.at[rows(qx, c)],
                 a_s.at[c], a_r.at[c], dev_x).wait_recv()
            if c == CZ_CHUNK:
                r = copy(other_vm.at[rows(qx, c)], other_vm.at[rows(qx, c)],
                         cz_s.at[0], cz_r.at[0], dev_z)
                r.start()
                sends.append(r)
            compute_block(qx * QROWS + c * CH)
            copy(other_vm.at[rows(qz, c)], other_vm.at[rows(qz, c)],
                 b_s.at[c], b_r.at[c], dev_z).wait_recv()
            if c == CX_CHUNK:
                r = copy(other_vm.at[rows(qz, c)], other_vm.at[rows(qz, c)],
                         cx_s.at[0], cx_r.at[0], dev_x)
                r.start()
                sends.append(r)
            compute_block(qz * QROWS + c * CH)

        for c in range(N_YDIR):
            y_rdmas[N_CH + c].wait_recv()
            compute_block(qd * QROWS + c * CH)
        copy(other_vm.at[rows(qd, CX_CHUNK)], other_vm.at[rows(qd, CX_CHUNK)],
             cx_s.at[0], cx_r.at[0], dev_x).wait_recv()
        compute_block(qd * QROWS + CX_CHUNK * CH)
        copy(other_vm.at[rows(qd, CZ_CHUNK)], other_vm.at[rows(qd, CZ_CHUNK)],
             cz_s.at[0], cz_r.at[0], dev_z).wait_recv()
        compute_block(qd * QROWS + CZ_CHUNK * CH)

        if state["pending"] is not None:
            _flush(state["pending"])
        for cpo in out_pending:
            if cpo is not None:
                cpo.wait()
        for r in sends:
            r.wait_send()

    dma = pltpu.SemaphoreType.DMA
    hbm = pltpu.MemorySpace.HBM
    return pl.pallas_call(
        body,
        out_shape=jax.ShapeDtypeStruct((M, D), jnp.float32),
        in_specs=[
            pl.BlockSpec(memory_space=hbm),
            pl.BlockSpec(memory_space=hbm),
            pl.BlockSpec(memory_space=pltpu.MemorySpace.VMEM),
        ],
        out_specs=pl.BlockSpec(memory_space=hbm),
        scratch_shapes=[
            pltpu.VMEM((M, D), jnp.bfloat16),
            dma((N_CH + N_YDIR,)), dma((N_CH + N_YDIR,)),
            dma((N_CH,)), dma((N_CH,)),
            dma((N_CH,)), dma((N_CH,)),
            dma((1,)), dma((1,)),
            dma((1,)), dma((1,)),
            pltpu.VMEM((2, CH, D), jnp.float32),
            pltpu.VMEM((2, CH, D), jnp.float32),
            dma((2,)), dma((2,)),
        ],
        compiler_params=pltpu.CompilerParams(
            collective_id=0,
            has_side_effects=True,
            vmem_limit_bytes=64 * 1024 * 1024,
        ),
    )(mysend_bf, slab, gamma)


def kernel(partial, resid, gamma):
    slab = partial[0]
    x = lax.axis_index("x")
    z = lax.axis_index("z")
    qm = 2 * x + z
    qd = 2 * (1 - x) + (1 - z)
    mysend_bf = jnp.concatenate(
        [
            lax.dynamic_slice(slab, (qm * QROWS, 0), (QROWS, D))
            + lax.dynamic_slice(resid, (qm * QROWS, 0), (QROWS, D)),
            lax.dynamic_slice(slab, (qd * QROWS, 0), (N_YDIR * CH, D))
            + lax.dynamic_slice(resid, (qd * QROWS, 0), (N_YDIR * CH, D)),
        ],
        axis=0,
    ).astype(jnp.bfloat16)
    return _fused(mysend_bf, slab, gamma.reshape(1, D))


# device time: 209787 ns/iter; 1.1058x vs baseline; 1.1058x over previous
import jax
import jax.numpy as jnp
from jax import lax
from jax.experimental import pallas as pl
from jax.experimental.pallas import tpu as pltpu

M = 8192
D = 2048
QROWS = M // 4
N_CH = 4
CH = QROWS // N_CH
N_YDIR = 2
CX_CHUNK = 2
CZ_CHUNK = 3


def _fused(slab, resid, qd_direct, gamma):

    def body(slab_ref, resid_ref, qddir_ref, gamma_ref, out_ref,
             other_vm, send_v,
             y_s, y_r, a_s, a_r, b_s, b_r, cx_s, cx_r, cz_s, cz_r,
             slab_v, out_v, dsem_s, dsem_w):
        x = lax.axis_index("x")
        yy = lax.axis_index("y")
        z = lax.axis_index("z")
        dev_y = (x, 1 - yy, z)
        dev_x = (1 - x, yy, z)
        dev_z = (x, yy, 1 - z)
        qm = 2 * x + z
        qx = 2 * (1 - x) + z
        qz = 2 * x + (1 - z)
        qd = 2 * (1 - x) + (1 - z)

        barrier = pltpu.get_barrier_semaphore()
        for d in (dev_y, dev_x, dev_z):
            pl.semaphore_signal(
                barrier, inc=1, device_id=d,
                device_id_type=pl.DeviceIdType.MESH,
            )
        pl.semaphore_wait(barrier, 3)

        def rows(q, c):
            return pl.ds(q * QROWS + c * CH, CH)

        def copy(src, dst, ssem, rsem, dev):
            return pltpu.make_async_remote_copy(
                src_ref=src, dst_ref=dst, send_sem=ssem, recv_sem=rsem,
                device_id=dev, device_id_type=pl.DeviceIdType.MESH,
            )

        state = {"pending": None, "slot": 0}
        out_pending = [None, None]

        def _stage(off):
            s = state["slot"]
            state["slot"] = 1 - s
            cp = pltpu.make_async_copy(slab_ref.at[pl.ds(off, CH)],
                                       slab_v.at[s], dsem_s.at[s])
            cp.start()
            return (off, cp, s)

        def _flush(p):
            off, cp, s = p
            cp.wait()
            if out_pending[s] is not None:
                out_pending[s].wait()
            yv = slab_v[s] + other_vm[pl.ds(off, CH)].astype(jnp.float32)
            ms = (yv * yv).sum(axis=1, keepdims=True) * (1.0 / D) + 1e-6
            out_v[s] = yv * lax.rsqrt(ms) * gamma_ref[...]
            cpo = pltpu.make_async_copy(out_v.at[s],
                                        out_ref.at[pl.ds(off, CH)],
                                        dsem_w.at[s])
            cpo.start()
            out_pending[s] = cpo

        def compute_block(off):
            p = _stage(off)
            if state["pending"] is not None:
                _flush(state["pending"])
            state["pending"] = p

        sends = []

        y_rdmas = []
        for c in range(N_CH):
            cp1 = pltpu.make_async_copy(slab_ref.at[rows(qm, c)],
                                        slab_v.at[0], dsem_s.at[0])
            cp2 = pltpu.make_async_copy(resid_ref.at[rows(qm, c)],
                                        slab_v.at[1], dsem_s.at[1])
            cp1.start()
            cp2.start()
            cp1.wait()
            cp2.wait()
            send_v[c] = (slab_v[0] + slab_v[1]).astype(jnp.bfloat16)
            r = copy(send_v.at[c], other_vm.at[rows(qm, c)],
                     y_s.at[c], y_r.at[c], dev_y)
            r.start()
            y_rdmas.append(r)
            sends.append(r)
        for c in range(N_YDIR):
            r = copy(qddir_ref.at[pl.ds(c * CH, CH)],
                     other_vm.at[rows(qd, c)],
                     y_s.at[N_CH + c], y_r.at[N_CH + c], dev_y)
            r.start()
            y_rdmas.append(r)
            sends.append(r)

        for c in range(N_CH):
            y_rdmas[c].wait_recv()
            for dev, ss, rr in ((dev_x, a_s, a_r), (dev_z, b_s, b_r)):
                r = copy(other_vm.at[rows(qm, c)], other_vm.at[rows(qm, c)],
                         ss.at[c], rr.at[c], dev)
                r.start()
                sends.append(r)
            compute_block(qm * QROWS + c * CH)

        for c in range(N_CH):
            copy(other_vm.at[rows(qx, c)], other_vm.at[rows(qx, c)],
                 a_s.at[c], a_r.at[c], dev_x).wait_recv()
            if c == CZ_CHUNK:
                r = copy(other_vm.at[rows(qx, c)], other_vm.at[rows(qx, c)],
                         cz_s.at[0], cz_r.at[0], dev_z)
                r.start()
                sends.append(r)
            compute_block(qx * QROWS + c * CH)
            copy(other_vm.at[rows(qz, c)], other_vm.at[rows(qz, c)],
                 b_s.at[c], b_r.at[c], dev_z).wait_recv()
            if c == CX_CHUNK:
                r = copy(other_vm.at[rows(qz, c)], other_vm.at[rows(qz, c)],
                         cx_s.at[0], cx_r.at[0], dev_x)
                r.start()
                sends.append(r)
            compute_block(qz * QROWS + c * CH)

        for c in range(N_YDIR):
            y_rdmas[N_CH + c].wait_recv()
            compute_block(qd * QROWS + c * CH)
        copy(other_vm.at[rows(qd, CX_CHUNK)], other_vm.at[rows(qd, CX_CHUNK)],
             cx_s.at[0], cx_r.at[0], dev_x).wait_recv()
        compute_block(qd * QROWS + CX_CHUNK * CH)
        copy(other_vm.at[rows(qd, CZ_CHUNK)], other_vm.at[rows(qd, CZ_CHUNK)],
             cz_s.at[0], cz_r.at[0], dev_z).wait_recv()
        compute_block(qd * QROWS + CZ_CHUNK * CH)

        if state["pending"] is not None:
            _flush(state["pending"])
        for cpo in out_pending:
            if cpo is not None:
                cpo.wait()
        for r in sends:
            r.wait_send()

    dma = pltpu.SemaphoreType.DMA
    hbm = pltpu.MemorySpace.HBM
    return pl.pallas_call(
        body,
        out_shape=jax.ShapeDtypeStruct((M, D), jnp.float32),
        in_specs=[
            pl.BlockSpec(memory_space=hbm),
            pl.BlockSpec(memory_space=hbm),
            pl.BlockSpec(memory_space=hbm),
            pl.BlockSpec(memory_space=pltpu.MemorySpace.VMEM),
        ],
        out_specs=pl.BlockSpec(memory_space=hbm),
        scratch_shapes=[
            pltpu.VMEM((M, D), jnp.bfloat16),
            pltpu.VMEM((N_CH, CH, D), jnp.bfloat16),
            dma((N_CH + N_YDIR,)), dma((N_CH + N_YDIR,)),
            dma((N_CH,)), dma((N_CH,)),
            dma((N_CH,)), dma((N_CH,)),
            dma((1,)), dma((1,)),
            dma((1,)), dma((1,)),
            pltpu.VMEM((2, CH, D), jnp.float32),
            pltpu.VMEM((2, CH, D), jnp.float32),
            dma((2,)), dma((2,)),
        ],
        compiler_params=pltpu.CompilerParams(
            collective_id=0,
            has_side_effects=True,
            vmem_limit_bytes=63 * 1024 * 1024,
        ),
    )(slab, resid, qd_direct, gamma)


def kernel(partial, resid, gamma):
    slab = partial[0]
    x = lax.axis_index("x")
    z = lax.axis_index("z")
    qd = 2 * (1 - x) + (1 - z)
    qd_direct = (
        lax.dynamic_slice(slab, (qd * QROWS, 0), (N_YDIR * CH, D))
        + lax.dynamic_slice(resid, (qd * QROWS, 0), (N_YDIR * CH, D))
    ).astype(jnp.bfloat16)
    return _fused(slab, resid, qd_direct, gamma.reshape(1, D))


# device time: 207240 ns/iter; 1.1194x vs baseline; 1.0123x over previous
import jax
import jax.numpy as jnp
from jax import lax
from jax.experimental import pallas as pl
from jax.experimental.pallas import tpu as pltpu

M = 8192
D = 2048
QROWS = M // 4
N_CH = 8
CH = QROWS // N_CH
N_YDIR = N_CH // 2
CX_CHUNKS = tuple(range(N_CH // 2, 3 * N_CH // 4))
CZ_CHUNKS = tuple(range(3 * N_CH // 4, N_CH))


def _fused(slab, resid, qd_direct, gamma):

    def body(slab_ref, resid_ref, qddir_ref, gamma_ref, out_ref,
             other_vm, send_v,
             y_s, y_r, a_s, a_r, b_s, b_r, cx_s, cx_r, cz_s, cz_r,
             slab_v, out_v, dsem_s, dsem_w):
        x = lax.axis_index("x")
        yy = lax.axis_index("y")
        z = lax.axis_index("z")
        dev_y = (x, 1 - yy, z)
        dev_x = (1 - x, yy, z)
        dev_z = (x, yy, 1 - z)
        qm = 2 * x + z
        qx = 2 * (1 - x) + z
        qz = 2 * x + (1 - z)
        qd = 2 * (1 - x) + (1 - z)

        barrier = pltpu.get_barrier_semaphore()
        for d in (dev_y, dev_x, dev_z):
            pl.semaphore_signal(
                barrier, inc=1, device_id=d,
                device_id_type=pl.DeviceIdType.MESH,
            )
        pl.semaphore_wait(barrier, 3)

        def rows(q, c):
            return pl.ds(q * QROWS + c * CH, CH)

        def copy(src, dst, ssem, rsem, dev):
            return pltpu.make_async_remote_copy(
                src_ref=src, dst_ref=dst, send_sem=ssem, recv_sem=rsem,
                device_id=dev, device_id_type=pl.DeviceIdType.MESH,
            )

        state = {"pending": None, "slot": 0}
        out_pending = [None, None]

        def _stage(off):
            s = state["slot"]
            state["slot"] = 1 - s
            cp = pltpu.make_async_copy(slab_ref.at[pl.ds(off, CH)],
                                       slab_v.at[s], dsem_s.at[s])
            cp.start()
            return (off, cp, s)

        def _flush(p):
            off, cp, s = p
            cp.wait()
            if out_pending[s] is not None:
                out_pending[s].wait()
            yv = slab_v[s] + other_vm[pl.ds(off, CH)].astype(jnp.float32)
            ms = (yv * yv).sum(axis=1, keepdims=True) * (1.0 / D) + 1e-6
            out_v[s] = yv * lax.rsqrt(ms) * gamma_ref[...]
            cpo = pltpu.make_async_copy(out_v.at[s],
                                        out_ref.at[pl.ds(off, CH)],
                                        dsem_w.at[s])
            cpo.start()
            out_pending[s] = cpo

        def compute_block(off):
            p = _stage(off)
            if state["pending"] is not None:
                _flush(state["pending"])
            state["pending"] = p

        sends = []

        y_rdmas = []
        for c in range(N_CH):
            cp1 = pltpu.make_async_copy(slab_ref.at[rows(qm, c)],
                                        slab_v.at[0], dsem_s.at[0])
            cp2 = pltpu.make_async_copy(resid_ref.at[rows(qm, c)],
                                        slab_v.at[1], dsem_s.at[1])
            cp1.start()
            cp2.start()
            cp1.wait()
            cp2.wait()
            send_v[c] = (slab_v[0] + slab_v[1]).astype(jnp.bfloat16)
            r = copy(send_v.at[c], other_vm.at[rows(qm, c)],
                     y_s.at[c], y_r.at[c], dev_y)
            r.start()
            y_rdmas.append(r)
            sends.append(r)
        for c in range(N_YDIR):
            r = copy(qddir_ref.at[pl.ds(c * CH, CH)],
                     other_vm.at[rows(qd, c)],
                     y_s.at[N_CH + c], y_r.at[N_CH + c], dev_y)
            r.start()
            y_rdmas.append(r)
            sends.append(r)

        for c in range(N_CH):
            y_rdmas[c].wait_recv()
            for dev, ss, rr in ((dev_x, a_s, a_r), (dev_z, b_s, b_r)):
                r = copy(other_vm.at[rows(qm, c)], other_vm.at[rows(qm, c)],
                         ss.at[c], rr.at[c], dev)
                r.start()
                sends.append(r)
            compute_block(qm * QROWS + c * CH)

        for c in range(N_CH):
            copy(other_vm.at[rows(qx, c)], other_vm.at[rows(qx, c)],
                 a_s.at[c], a_r.at[c], dev_x).wait_recv()
            if c in CZ_CHUNKS:
                i = c - CZ_CHUNKS[0]
                r = copy(other_vm.at[rows(qx, c)], other_vm.at[rows(qx, c)],
                         cz_s.at[i], cz_r.at[i], dev_z)
                r.start()
                sends.append(r)
            compute_block(qx * QROWS + c * CH)
            copy(other_vm.at[rows(qz, c)], other_vm.at[rows(qz, c)],
                 b_s.at[c], b_r.at[c], dev_z).wait_recv()
            if c in CX_CHUNKS:
                i = c - CX_CHUNKS[0]
                r = copy(other_vm.at[rows(qz, c)], other_vm.at[rows(qz, c)],
                         cx_s.at[i], cx_r.at[i], dev_x)
                r.start()
                sends.append(r)
            compute_block(qz * QROWS + c * CH)

        for c in range(N_YDIR):
            y_rdmas[N_CH + c].wait_recv()
            compute_block(qd * QROWS + c * CH)
        for i, c in enumerate(CX_CHUNKS):
            copy(other_vm.at[rows(qd, c)], other_vm.at[rows(qd, c)],
                 cx_s.at[i], cx_r.at[i], dev_x).wait_recv()
            compute_block(qd * QROWS + c * CH)
        for i, c in enumerate(CZ_CHUNKS):
            copy(other_vm.at[rows(qd, c)], other_vm.at[rows(qd, c)],
                 cz_s.at[i], cz_r.at[i], dev_z).wait_recv()
            compute_block(qd * QROWS + c * CH)

        if state["pending"] is not None:
            _flush(state["pending"])
        for cpo in out_pending:
            if cpo is not None:
                cpo.wait()
        for r in sends:
            r.wait_send()

    dma = pltpu.SemaphoreType.DMA
    hbm = pltpu.MemorySpace.HBM
    return pl.pallas_call(
        body,
        out_shape=jax.ShapeDtypeStruct((M, D), jnp.float32),
        in_specs=[
            pl.BlockSpec(memory_space=hbm),
            pl.BlockSpec(memory_space=hbm),
            pl.BlockSpec(memory_space=hbm),
            pl.BlockSpec(memory_space=pltpu.MemorySpace.VMEM),
        ],
        out_specs=pl.BlockSpec(memory_space=hbm),
        scratch_shapes=[
            pltpu.VMEM((M, D), jnp.bfloat16),
            pltpu.VMEM((N_CH, CH, D), jnp.bfloat16),
            dma((N_CH + N_YDIR,)), dma((N_CH + N_YDIR,)),
            dma((N_CH,)), dma((N_CH,)),
            dma((N_CH,)), dma((N_CH,)),
            dma((len(CX_CHUNKS),)), dma((len(CX_CHUNKS),)),
            dma((len(CZ_CHUNKS),)), dma((len(CZ_CHUNKS),)),
            pltpu.VMEM((2, CH, D), jnp.float32),
            pltpu.VMEM((2, CH, D), jnp.float32),
            dma((2,)), dma((2,)),
        ],
        compiler_params=pltpu.CompilerParams(
            collective_id=0,
            has_side_effects=True,
            vmem_limit_bytes=63 * 1024 * 1024,
        ),
    )(slab, resid, qd_direct, gamma)


def kernel(partial, resid, gamma):
    slab = partial[0]
    x = lax.axis_index("x")
    z = lax.axis_index("z")
    qd = 2 * (1 - x) + (1 - z)
    qd_direct = (
        lax.dynamic_slice(slab, (qd * QROWS, 0), (N_YDIR * CH, D))
        + lax.dynamic_slice(resid, (qd * QROWS, 0), (N_YDIR * CH, D))
    ).astype(jnp.bfloat16)
    return _fused(slab, resid, qd_direct, gamma.reshape(1, D))
